# baseline (speedup 1.0000x reference)
import jax
import jax.numpy as jnp
from jax import lax
from jax.experimental import pallas as pl
from jax.experimental.pallas import tpu as pltpu
from jax.experimental.pallas import tpu_sc as plsc

NC = 2
NS = 16
LANES = 16
NW = NC * NS

LDEG = 16
ZB = 80


K = 80
NSLOT = 6
LOOKAHEAD = 3


def _make_sc_aggregate(n, d, e, with_deg):
    dh = d // NC
    ew = e // NS
    k = K
    nch = ew // k
    nb = n // ZB
    tmax = -(-nb // NS)

    mesh = plsc.VectorSubcoreMesh(core_axis_name="c", subcore_axis_name="s")

    out_type = [jax.ShapeDtypeStruct((NC, n, dh), jnp.float32)]
    scratch = [
        pltpu.VMEM((nch, k), jnp.int32),
        pltpu.VMEM((nch, k), jnp.int32),
        pltpu.VMEM((NSLOT, k, dh), jnp.float32),
        pltpu.VMEM((ZB, dh), jnp.float32),
        pltpu.SemaphoreType.DMA((NSLOT,)),
        pltpu.SemaphoreType.DMA((NSLOT,)),
        pltpu.VMEM_SHARED((n + 8, dh), jnp.float32),
    ]
    if with_deg:
        out_type.append(jax.ShapeDtypeStruct((NC, n, LDEG), jnp.float32))
        scratch += [
            pltpu.VMEM((k, LDEG), jnp.float32),
            pltpu.VMEM((ZB, LDEG), jnp.float32),
            pltpu.VMEM_SHARED((n + 8, LDEG), jnp.float32),
        ]

    def body(table, srcw, dstw, *refs):
        if with_deg:
            (out, dego, srcv, dstv, bufs, zbuf, gsem, ssem, acc,
             ones, zdeg, dacc) = refs
        else:
            out, srcv, dstv, bufs, zbuf, gsem, ssem, acc = refs
        c = lax.axis_index("c")
        s = lax.axis_index("s")
        wid = c * NS + s
        z16 = jnp.zeros((LANES,), jnp.float32)

        @pl.loop(0, ZB)
        def _(r):
            @pl.loop(0, dh, step=LANES)
            def _(cc):
                zbuf[r, pl.ds(cc, LANES)] = z16

        @pl.loop(0, tmax)
        def _(t):
            bid = s + t * NS

            @pl.when(bid < nb)
            def _():
                pltpu.sync_copy(zbuf, acc.at[pl.ds(bid * ZB, ZB)])

        if with_deg:
            o16 = jnp.ones((LANES,), jnp.float32)

            @pl.loop(0, ZB)
            def _(r):
                zdeg[r, pl.ds(0, LDEG)] = z16

            @pl.loop(0, k)
            def _(r):
                ones[r, pl.ds(0, LDEG)] = o16

            @pl.loop(0, tmax)
            def _(t):
                bid = s + t * NS

                @pl.when(bid < nb)
                def _():
                    pltpu.sync_copy(zdeg, dacc.at[pl.ds(bid * ZB, ZB)])

        pltpu.sync_copy(srcw.at[wid], srcv)
        pltpu.sync_copy(dstw.at[s], dstv)

        plsc.subcore_barrier()

        def start_gather(i, b):
            pltpu.async_copy(table.at[srcv.at[i]], bufs.at[b], gsem.at[b])

        def wait_gather(i, b):
            pltpu.make_async_copy(table.at[srcv.at[i]], bufs.at[b],
                                  gsem.at[b]).wait()

        def start_scatter(i, b):
            pltpu.async_copy(bufs.at[b], acc.at[dstv.at[i]], ssem.at[b],
                             add=True)

        def wait_scatter(i, b):
            pltpu.make_async_copy(bufs.at[b], acc.at[dstv.at[i]],
                                  ssem.at[b]).wait()

        for b in range(LOOKAHEAD):
            start_gather(b, b)

        @pl.loop(0, nch, step=NSLOT)
        def _(j):
            for b in range(NSLOT):
                i = j + b
                bl = (b + LOOKAHEAD) % NSLOT

                @pl.when(i + LOOKAHEAD < nch)
                def _():
                    @pl.when(i + LOOKAHEAD - NSLOT >= 0)
                    def _():
                        wait_scatter(i + LOOKAHEAD - NSLOT, bl)

                    start_gather(i + LOOKAHEAD, bl)

                wait_gather(i, b)
                start_scatter(i, b)
                if with_deg:
                    @pl.when(lax.rem(i, 2) == c)
                    def _():
                        pltpu.sync_copy(ones, dacc.at[dstv.at[i]], add=True)

        for b in range(NSLOT):
            i = nch - NSLOT + b
            wait_scatter(i, b)

        plsc.subcore_barrier()

        @pl.loop(0, tmax)
        def _(t):
            bid = s + t * NS

            @pl.when(bid < nb)
            def _():
                pltpu.sync_copy(acc.at[pl.ds(bid * ZB, ZB)],
                                out.at[c, pl.ds(bid * ZB, ZB)])

        if with_deg:
            @pl.loop(0, tmax)
            def _(t):
                bid = s + t * NS

                @pl.when(bid < nb)
                def _():
                    pltpu.sync_copy(dacc.at[pl.ds(bid * ZB, ZB)],
                                    dego.at[c, pl.ds(bid * ZB, ZB)])

    return pl.kernel(
        body, out_type=out_type, mesh=mesh, scratch_types=scratch,
        compiler_params=pltpu.CompilerParams(use_tc_tiling_on_sc=False))


RB = 2000


def _mm_in(x, wcat, bcat, d):
    n = x.shape[0]
    d_in = x.shape[1]
    dh = d // NC

    def body(x_ref, w_ref, b_ref, o1_ref, o2_ref):
        h = jnp.dot(x_ref[...], w_ref[...],
                    preferred_element_type=jnp.float32,
                    precision=lax.Precision.HIGHEST) + b_ref[...]
        o1_ref[0] = h[:, :dh]
        o1_ref[1] = h[:, dh:d]
        o2_ref[...] = h[:, d:]

    return pl.pallas_call(
        body,
        grid=(n // RB,),
        in_specs=[pl.BlockSpec((RB, d_in), lambda i: (i, 0)),
                  pl.BlockSpec(wcat.shape, lambda i: (0, 0)),
                  pl.BlockSpec(bcat.shape, lambda i: (0, 0))],
        out_specs=[pl.BlockSpec((NC, RB, dh), lambda i: (0, i, 0)),
                   pl.BlockSpec((RB, d), lambda i: (i, 0))],
        out_shape=[jax.ShapeDtypeStruct((NC, n, dh), jnp.float32),
                   jax.ShapeDtypeStruct((n, d), jnp.float32)],
    )(x, wcat, bcat)


def _mid(aggp, degp, xr, wcat, bcat, d):
    n, d_in = xr.shape
    dha = aggp.shape[2]
    dh = d // NC

    def body(a_ref, g_ref, xr_ref, w_ref, b_ref, o1_ref, o2_ref):
        agg = jnp.concatenate([a_ref[0], a_ref[1]], axis=1)
        deg = (jnp.max(g_ref[0], axis=1, keepdims=True)
               + jnp.max(g_ref[1], axis=1, keepdims=True))
        dinv = 1.0 / jnp.maximum(deg, 1.0)
        h = jnp.maximum(agg * dinv + xr_ref[...], 0.0)
        hcat = jnp.dot(h, w_ref[...],
                       preferred_element_type=jnp.float32,
                       precision=lax.Precision.HIGHEST) + b_ref[...]
        o1_ref[0] = hcat[:, :dh]
        o1_ref[1] = hcat[:, dh:d]
        o2_ref[...] = hcat[:, d:]

    return pl.pallas_call(
        body,
        grid=(n // RB,),
        in_specs=[pl.BlockSpec((NC, RB, dha), lambda i: (0, i, 0)),
                  pl.BlockSpec((NC, RB, LDEG), lambda i: (0, i, 0)),
                  pl.BlockSpec((RB, d_in), lambda i: (i, 0)),
                  pl.BlockSpec(wcat.shape, lambda i: (0, 0)),
                  pl.BlockSpec(bcat.shape, lambda i: (0, 0))],
        out_specs=[pl.BlockSpec((NC, RB, dh), lambda i: (0, i, 0)),
                   pl.BlockSpec((RB, d), lambda i: (i, 0))],
        out_shape=[jax.ShapeDtypeStruct((NC, n, dh), jnp.float32),
                   jax.ShapeDtypeStruct((n, d), jnp.float32)],
    )(aggp, degp, xr, wcat, bcat)


def _epilogue(aggp, degp, hr):
    n, d = hr.shape
    dha = aggp.shape[2]

    def body(a_ref, g_ref, hr_ref, o_ref):
        agg = jnp.concatenate([a_ref[0], a_ref[1]], axis=1)
        deg = (jnp.max(g_ref[0], axis=1, keepdims=True)
               + jnp.max(g_ref[1], axis=1, keepdims=True))
        dinv = 1.0 / jnp.maximum(deg, 1.0)
        o_ref[...] = agg * dinv + hr_ref[...]

    return pl.pallas_call(
        body,
        grid=(n // RB,),
        in_specs=[pl.BlockSpec((NC, RB, dha), lambda i: (0, i, 0)),
                  pl.BlockSpec((NC, RB, LDEG), lambda i: (0, i, 0)),
                  pl.BlockSpec((RB, d), lambda i: (i, 0))],
        out_specs=pl.BlockSpec((RB, d), lambda i: (i, 0)),
        out_shape=jax.ShapeDtypeStruct(hr.shape, jnp.float32),
    )(aggp, degp, hr)


def kernel(x, edge_index, W1_l, W1_r, b1, W2_l, W2_r, b2):
    n, d_in = x.shape
    e = edge_index.shape[1]
    d_hid = W1_l.shape[0]
    d_out = W2_l.shape[0]

    ei = edge_index.astype(jnp.int32)
    grp = K * NSLOT
    ewp = -(-(-(-e // NS)) // grp) * grp
    ep = ewp * NS
    pad = ep - e
    src_flat = jnp.concatenate([ei[0], jnp.zeros((pad,), jnp.int32)])
    dst_flat = jnp.concatenate([ei[1], jnp.full((pad,), n, jnp.int32)])
    nch = ewp // K
    src2 = src_flat.reshape(NS, nch, K)
    srcw = jnp.concatenate([src2, src2 + n], axis=0)
    dstw = dst_flat.reshape(NS, nch, K)

    w1cat = jnp.concatenate([W1_l.T, W1_r.T], axis=1)
    b1cat = jnp.concatenate([jnp.zeros_like(b1), b1]).reshape(1, 2 * d_hid)
    w2cat = jnp.concatenate([W2_l.T, W2_r.T], axis=1)
    b2cat = jnp.concatenate([jnp.zeros_like(b2), b2]).reshape(1, 2 * d_out)

    agg1 = _make_sc_aggregate(n, d_hid, ep, with_deg=True)
    agg2 = _make_sc_aggregate(n, d_out, ep, with_deg=False)

    xl, xr = _mm_in(x, w1cat, b1cat, d_hid)
    aggp1, degp = agg1(xl.reshape(NC * n, d_hid // NC), srcw, dstw)
    hl, hr = _mid(aggp1, degp, xr, w2cat, b2cat, d_out)
    aggp2 = agg2(hl.reshape(NC * n, d_out // NC), srcw, dstw)
    if isinstance(aggp2, (list, tuple)):
        aggp2 = aggp2[0]
    return _epilogue(aggp2, degp, hr)


# speedup vs baseline: 1.0550x; 1.0550x over previous
import jax
import jax.numpy as jnp
from jax import lax
from jax.experimental import pallas as pl
from jax.experimental.pallas import tpu as pltpu
from jax.experimental.pallas import tpu_sc as plsc

NC = 2
NS = 16
LANES = 16
NW = NC * NS

LDEG = 16
ZB = 80


K = 80
NSLOT = 6
LOOKAHEAD = 3


def _make_sc_aggregate(n, d, e, with_deg):
    dh = d // NC
    ew = e // NS
    k = K
    nch = ew // k
    nb = n // ZB
    tmax = -(-nb // NS)

    mesh = plsc.VectorSubcoreMesh(core_axis_name="c", subcore_axis_name="s")

    out_type = [jax.ShapeDtypeStruct((NC, n, dh), jnp.float32)]
    scratch = [
        pltpu.VMEM((nch, k), jnp.int32),
        pltpu.VMEM((nch, k), jnp.int32),
        pltpu.VMEM((NSLOT, k, dh), jnp.float32),
        pltpu.VMEM((ZB, dh), jnp.float32),
        pltpu.SemaphoreType.DMA((NSLOT,)),
        pltpu.SemaphoreType.DMA((NSLOT,)),
        pltpu.VMEM_SHARED((n + 8, dh), jnp.float32),
    ]
    if with_deg:
        out_type.append(jax.ShapeDtypeStruct((NC, n, LDEG), jnp.float32))
        scratch += [
            pltpu.VMEM((k, LDEG), jnp.float32),
            pltpu.VMEM((ZB, LDEG), jnp.float32),
            pltpu.VMEM_SHARED((n + 8, LDEG), jnp.float32),
        ]

    def body(table, srcw, dstw, *refs):
        if with_deg:
            (out, dego, srcv, dstv, bufs, zbuf, gsem, ssem, acc,
             ones, zdeg, dacc) = refs
        else:
            out, srcv, dstv, bufs, zbuf, gsem, ssem, acc = refs
        c = lax.axis_index("c")
        s = lax.axis_index("s")
        wid = c * NS + s
        z16 = jnp.zeros((LANES,), jnp.float32)

        @pl.loop(0, ZB)
        def _(r):
            @pl.loop(0, dh, step=LANES)
            def _(cc):
                zbuf[r, pl.ds(cc, LANES)] = z16

        @pl.loop(0, tmax)
        def _(t):
            bid = s + t * NS

            @pl.when(bid < nb)
            def _():
                pltpu.sync_copy(zbuf, acc.at[pl.ds(bid * ZB, ZB)])

        if with_deg:
            o16 = jnp.ones((LANES,), jnp.float32)

            @pl.loop(0, ZB)
            def _(r):
                zdeg[r, pl.ds(0, LDEG)] = z16

            @pl.loop(0, k)
            def _(r):
                ones[r, pl.ds(0, LDEG)] = o16

            @pl.loop(0, tmax)
            def _(t):
                bid = s + t * NS

                @pl.when(bid < nb)
                def _():
                    pltpu.sync_copy(zdeg, dacc.at[pl.ds(bid * ZB, ZB)])

        pltpu.sync_copy(srcw.at[wid], srcv)
        pltpu.sync_copy(dstw.at[s], dstv)

        plsc.subcore_barrier()

        def start_gather(i, b):
            pltpu.async_copy(table.at[srcv.at[i]], bufs.at[b], gsem.at[b])

        def wait_gather(i, b):
            pltpu.make_async_copy(table.at[srcv.at[i]], bufs.at[b],
                                  gsem.at[b]).wait()

        def start_scatter(i, b):
            pltpu.async_copy(bufs.at[b], acc.at[dstv.at[i]], ssem.at[b],
                             add=True)

        def wait_scatter(i, b):
            pltpu.make_async_copy(bufs.at[b], acc.at[dstv.at[i]],
                                  ssem.at[b]).wait()

        for b in range(LOOKAHEAD):
            start_gather(b, b)

        @pl.loop(0, nch, step=NSLOT)
        def _(j):
            for b in range(NSLOT):
                i = j + b
                bl = (b + LOOKAHEAD) % NSLOT

                @pl.when(i + LOOKAHEAD < nch)
                def _():
                    @pl.when(i + LOOKAHEAD - NSLOT >= 0)
                    def _():
                        wait_scatter(i + LOOKAHEAD - NSLOT, bl)

                    start_gather(i + LOOKAHEAD, bl)

                wait_gather(i, b)
                start_scatter(i, b)
                if with_deg:
                    @pl.when(lax.rem(i, 2) == c)
                    def _():
                        pltpu.sync_copy(ones, dacc.at[dstv.at[i]], add=True)

        for b in range(NSLOT):
            i = nch - NSLOT + b
            wait_scatter(i, b)

        plsc.subcore_barrier()

        @pl.loop(0, tmax)
        def _(t):
            bid = s + t * NS

            @pl.when(bid < nb)
            def _():
                pltpu.sync_copy(acc.at[pl.ds(bid * ZB, ZB)],
                                out.at[c, pl.ds(bid * ZB, ZB)])

        if with_deg:
            @pl.loop(0, tmax)
            def _(t):
                bid = s + t * NS

                @pl.when(bid < nb)
                def _():
                    pltpu.sync_copy(dacc.at[pl.ds(bid * ZB, ZB)],
                                    dego.at[c, pl.ds(bid * ZB, ZB)])

    return pl.kernel(
        body, out_type=out_type, mesh=mesh, scratch_types=scratch,
        compiler_params=pltpu.CompilerParams(use_tc_tiling_on_sc=False))


RB = 2000


def _mm_in(x, wcat, bcat, d):
    n = x.shape[0]
    d_in = x.shape[1]
    dh = d // NC

    def body(x_ref, w_ref, b_ref, o1_ref, o2_ref):
        h = jnp.dot(x_ref[...], w_ref[...],
                    preferred_element_type=jnp.float32,
                    precision=lax.Precision.DEFAULT) + b_ref[...]
        o1_ref[0] = h[:, :dh]
        o1_ref[1] = h[:, dh:d]
        o2_ref[...] = h[:, d:]

    return pl.pallas_call(
        body,
        grid=(n // RB,),
        in_specs=[pl.BlockSpec((RB, d_in), lambda i: (i, 0)),
                  pl.BlockSpec(wcat.shape, lambda i: (0, 0)),
                  pl.BlockSpec(bcat.shape, lambda i: (0, 0))],
        out_specs=[pl.BlockSpec((NC, RB, dh), lambda i: (0, i, 0)),
                   pl.BlockSpec((RB, d), lambda i: (i, 0))],
        out_shape=[jax.ShapeDtypeStruct((NC, n, dh), jnp.float32),
                   jax.ShapeDtypeStruct((n, d), jnp.float32)],
    )(x, wcat, bcat)


def _mid(aggp, degp, xr, wcat, bcat, d):
    n, d_in = xr.shape
    dha = aggp.shape[2]
    dh = d // NC

    def body(a_ref, g_ref, xr_ref, w_ref, b_ref, o1_ref, o2_ref):
        agg = jnp.concatenate([a_ref[0], a_ref[1]], axis=1)
        deg = (jnp.max(g_ref[0], axis=1, keepdims=True)
               + jnp.max(g_ref[1], axis=1, keepdims=True))
        dinv = 1.0 / jnp.maximum(deg, 1.0)
        h = jnp.maximum(agg * dinv + xr_ref[...], 0.0)
        hcat = jnp.dot(h, w_ref[...],
                       preferred_element_type=jnp.float32,
                       precision=lax.Precision.DEFAULT) + b_ref[...]
        o1_ref[0] = hcat[:, :dh]
        o1_ref[1] = hcat[:, dh:d]
        o2_ref[...] = hcat[:, d:]

    return pl.pallas_call(
        body,
        grid=(n // RB,),
        in_specs=[pl.BlockSpec((NC, RB, dha), lambda i: (0, i, 0)),
                  pl.BlockSpec((NC, RB, LDEG), lambda i: (0, i, 0)),
                  pl.BlockSpec((RB, d_in), lambda i: (i, 0)),
                  pl.BlockSpec(wcat.shape, lambda i: (0, 0)),
                  pl.BlockSpec(bcat.shape, lambda i: (0, 0))],
        out_specs=[pl.BlockSpec((NC, RB, dh), lambda i: (0, i, 0)),
                   pl.BlockSpec((RB, d), lambda i: (i, 0))],
        out_shape=[jax.ShapeDtypeStruct((NC, n, dh), jnp.float32),
                   jax.ShapeDtypeStruct((n, d), jnp.float32)],
    )(aggp, degp, xr, wcat, bcat)


def _epilogue(aggp, degp, hr):
    n, d = hr.shape
    dha = aggp.shape[2]

    def body(a_ref, g_ref, hr_ref, o_ref):
        agg = jnp.concatenate([a_ref[0], a_ref[1]], axis=1)
        deg = (jnp.max(g_ref[0], axis=1, keepdims=True)
               + jnp.max(g_ref[1], axis=1, keepdims=True))
        dinv = 1.0 / jnp.maximum(deg, 1.0)
        o_ref[...] = agg * dinv + hr_ref[...]

    return pl.pallas_call(
        body,
        grid=(n // RB,),
        in_specs=[pl.BlockSpec((NC, RB, dha), lambda i: (0, i, 0)),
                  pl.BlockSpec((NC, RB, LDEG), lambda i: (0, i, 0)),
                  pl.BlockSpec((RB, d), lambda i: (i, 0))],
        out_specs=pl.BlockSpec((RB, d), lambda i: (i, 0)),
        out_shape=jax.ShapeDtypeStruct(hr.shape, jnp.float32),
    )(aggp, degp, hr)


def kernel(x, edge_index, W1_l, W1_r, b1, W2_l, W2_r, b2):
    n, d_in = x.shape
    e = edge_index.shape[1]
    d_hid = W1_l.shape[0]
    d_out = W2_l.shape[0]

    ei = edge_index.astype(jnp.int32)
    grp = K * NSLOT
    ewp = -(-(-(-e // NS)) // grp) * grp
    ep = ewp * NS
    pad = ep - e
    src_flat = jnp.concatenate([ei[0], jnp.zeros((pad,), jnp.int32)])
    dst_flat = jnp.concatenate([ei[1], jnp.full((pad,), n, jnp.int32)])
    nch = ewp // K
    src2 = src_flat.reshape(NS, nch, K)
    srcw = jnp.concatenate([src2, src2 + n], axis=0)
    dstw = dst_flat.reshape(NS, nch, K)

    w1cat = jnp.concatenate([W1_l.T, W1_r.T], axis=1)
    b1cat = jnp.concatenate([jnp.zeros_like(b1), b1]).reshape(1, 2 * d_hid)
    w2cat = jnp.concatenate([W2_l.T, W2_r.T], axis=1)
    b2cat = jnp.concatenate([jnp.zeros_like(b2), b2]).reshape(1, 2 * d_out)

    agg1 = _make_sc_aggregate(n, d_hid, ep, with_deg=True)
    agg2 = _make_sc_aggregate(n, d_out, ep, with_deg=False)

    xl, xr = _mm_in(x, w1cat, b1cat, d_hid)
    aggp1, degp = agg1(xl.reshape(NC * n, d_hid // NC), srcw, dstw)
    hl, hr = _mid(aggp1, degp, xr, w2cat, b2cat, d_out)
    aggp2 = agg2(hl.reshape(NC * n, d_out // NC), srcw, dstw)
    if isinstance(aggp2, (list, tuple)):
        aggp2 = aggp2[0]
    return _epilogue(aggp2, degp, hr)


# speedup vs baseline: 1.4231x; 1.3489x over previous
import jax
import jax.numpy as jnp
from jax import lax
from jax.experimental import pallas as pl
from jax.experimental.pallas import tpu as pltpu
from jax.experimental.pallas import tpu_sc as plsc

NC = 2
NS = 16
LANES = 16
NW = NC * NS

LDEG = 16
ZB = 80


K = 80
NSLOT = 5
LOOKAHEAD = 3


def _make_sc_aggregate(n, d, e, with_deg, fused_tail):
    dh = d // NC
    ew = e // NS
    k = K
    nch = ew // k
    nb = n // ZB
    tmax = -(-nb // NS)

    mesh = plsc.VectorSubcoreMesh(core_axis_name="c", subcore_axis_name="s")

    if fused_tail:
        out_type = [jax.ShapeDtypeStruct((n, d), jnp.float32)]
    else:
        out_type = [jax.ShapeDtypeStruct((NC, n, dh), jnp.float32)]
    scratch = [
        pltpu.VMEM((nch, k), jnp.int32),
        pltpu.VMEM((nch, k), jnp.int32),
        pltpu.VMEM((NSLOT, k, dh), jnp.float32),
        pltpu.VMEM((ZB, dh), jnp.float32),
        pltpu.SemaphoreType.DMA((NSLOT,)),
        pltpu.SemaphoreType.DMA((NSLOT,)),
        pltpu.SemaphoreType.DMA((2,)),
        pltpu.VMEM_SHARED((n + 8, dh), jnp.float32),
    ]
    if with_deg:
        out_type.append(jax.ShapeDtypeStruct((NC, n, LDEG), jnp.float32))
        scratch += [
            pltpu.VMEM((k, LDEG), jnp.float32),
            pltpu.VMEM((ZB, LDEG), jnp.float32),
            pltpu.VMEM_SHARED((n + 8, LDEG), jnp.float32),
        ]
    if fused_tail:
        scratch += [
            pltpu.VMEM((ZB, dh), jnp.float32),
            pltpu.VMEM((ZB, LDEG), jnp.float32),
        ]

    def body(table, srcw, dstw, *refs):
        hrp = dinvp = None
        if fused_tail:
            table, srcw, dstw, hrp, dinvp = (table, srcw, dstw) + refs[:2]
            refs = refs[2:]
        if with_deg:
            (out, dego, srcv, dstv, bufs, zbuf, gsem, ssem, isem, acc,
             ones, zdeg, dacc) = refs
        else:
            if fused_tail:
                (out, srcv, dstv, bufs, zbuf, gsem, ssem, isem, acc,
                 hbuf, dvbuf) = refs
            else:
                out, srcv, dstv, bufs, zbuf, gsem, ssem, isem, acc = refs
        c = lax.axis_index("c")
        s = lax.axis_index("s")
        wid = c * NS + s
        z16 = jnp.zeros((LANES,), jnp.float32)

        pltpu.async_copy(srcw.at[wid], srcv, isem.at[0])
        pltpu.async_copy(dstw.at[s], dstv, isem.at[1])

        @pl.loop(0, ZB)
        def _(r):
            @pl.loop(0, dh, step=LANES)
            def _(cc):
                zbuf[r, pl.ds(cc, LANES)] = z16

        @pl.loop(0, tmax)
        def _(t):
            bid = s + t * NS

            @pl.when(bid < nb)
            def _():
                pltpu.sync_copy(zbuf, acc.at[pl.ds(bid * ZB, ZB)])

        if with_deg:
            o16 = jnp.ones((LANES,), jnp.float32)

            @pl.loop(0, ZB)
            def _(r):
                zdeg[r, pl.ds(0, LDEG)] = z16

            @pl.loop(0, k)
            def _(r):
                ones[r, pl.ds(0, LDEG)] = o16

            @pl.loop(0, tmax)
            def _(t):
                bid = s + t * NS

                @pl.when(bid < nb)
                def _():
                    pltpu.sync_copy(zdeg, dacc.at[pl.ds(bid * ZB, ZB)])

        pltpu.make_async_copy(srcw.at[wid], srcv, isem.at[0]).wait()
        pltpu.make_async_copy(dstw.at[s], dstv, isem.at[1]).wait()

        plsc.subcore_barrier()

        def start_gather(i, b):
            pltpu.async_copy(table.at[srcv.at[i]], bufs.at[b], gsem.at[b])

        def wait_gather(i, b):
            pltpu.make_async_copy(table.at[srcv.at[i]], bufs.at[b],
                                  gsem.at[b]).wait()

        def start_scatter(i, b):
            pltpu.async_copy(bufs.at[b], acc.at[dstv.at[i]], ssem.at[b],
                             add=True)

        def wait_scatter(i, b):
            pltpu.make_async_copy(bufs.at[b], acc.at[dstv.at[i]],
                                  ssem.at[b]).wait()

        for b in range(LOOKAHEAD):
            start_gather(b, b)

        @pl.loop(0, nch, step=NSLOT)
        def _(j):
            for b in range(NSLOT):
                i = j + b
                bl = (b + LOOKAHEAD) % NSLOT

                @pl.when(i + LOOKAHEAD < nch)
                def _():
                    @pl.when(i + LOOKAHEAD - NSLOT >= 0)
                    def _():
                        wait_scatter(i + LOOKAHEAD - NSLOT, bl)

                    start_gather(i + LOOKAHEAD, bl)

                wait_gather(i, b)
                start_scatter(i, b)
                if with_deg:
                    @pl.when(lax.rem(i, 2) == c)
                    def _():
                        pltpu.sync_copy(ones, dacc.at[dstv.at[i]], add=True)

        for b in range(NSLOT):
            i = nch - NSLOT + b
            wait_scatter(i, b)

        plsc.subcore_barrier()

        if fused_tail:
            @pl.loop(0, tmax)
            def _(t):
                bid = s + t * NS

                @pl.when(bid < nb)
                def _():
                    r0 = bid * ZB
                    pltpu.sync_copy(acc.at[pl.ds(r0, ZB)], zbuf)
                    pltpu.sync_copy(hrp.at[c, pl.ds(r0, ZB)], hbuf)
                    pltpu.sync_copy(dinvp.at[pl.ds(r0, ZB)], dvbuf)

                    @pl.loop(0, ZB)
                    def _(r):
                        dinv = dvbuf[r, pl.ds(0, LANES)]
                        for q in range(dh // LANES):
                            sl = pl.ds(q * LANES, LANES)
                            zbuf[r, sl] = zbuf[r, sl] * dinv + hbuf[r, sl]

                    pltpu.sync_copy(
                        zbuf, out.at[pl.ds(r0, ZB), pl.ds(c * dh, dh)])

        else:
            @pl.loop(0, tmax)
            def _(t):
                bid = s + t * NS

                @pl.when(bid < nb)
                def _():
                    pltpu.sync_copy(acc.at[pl.ds(bid * ZB, ZB)],
                                    out.at[c, pl.ds(bid * ZB, ZB)])

        if with_deg:
            @pl.loop(0, tmax)
            def _(t):
                bid = s + t * NS

                @pl.when(bid < nb)
                def _():
                    pltpu.sync_copy(dacc.at[pl.ds(bid * ZB, ZB)],
                                    dego.at[c, pl.ds(bid * ZB, ZB)])

    return pl.kernel(
        body, out_type=out_type, mesh=mesh, scratch_types=scratch,
        compiler_params=pltpu.CompilerParams(use_tc_tiling_on_sc=False))


RB = 2000


def _mm_in(x, wcat, bcat, d):
    n = x.shape[0]
    d_in = x.shape[1]
    dh = d // NC

    def body(x_ref, w_ref, b_ref, o1_ref, o2_ref):
        h = jnp.dot(x_ref[...], w_ref[...],
                    preferred_element_type=jnp.float32,
                    precision=lax.Precision.DEFAULT) + b_ref[...]
        o1_ref[0] = h[:, :dh]
        o1_ref[1] = h[:, dh:d]
        o2_ref[...] = h[:, d:]

    return pl.pallas_call(
        body,
        grid=(n // RB,),
        in_specs=[pl.BlockSpec((RB, d_in), lambda i: (i, 0)),
                  pl.BlockSpec(wcat.shape, lambda i: (0, 0)),
                  pl.BlockSpec(bcat.shape, lambda i: (0, 0))],
        out_specs=[pl.BlockSpec((NC, RB, dh), lambda i: (0, i, 0)),
                   pl.BlockSpec((RB, d), lambda i: (i, 0))],
        out_shape=[jax.ShapeDtypeStruct((NC, n, dh), jnp.float32),
                   jax.ShapeDtypeStruct((n, d), jnp.float32)],
    )(x, wcat, bcat)


def _mid(aggp, degp, xr, wcat, bcat, d):
    n, d_in = xr.shape
    dha = aggp.shape[2]
    dh = d // NC

    def body(a_ref, g_ref, xr_ref, w_ref, b_ref, o1_ref, o2_ref, o3_ref):
        agg = jnp.concatenate([a_ref[0], a_ref[1]], axis=1)
        deg = (jnp.max(g_ref[0], axis=1, keepdims=True)
               + jnp.max(g_ref[1], axis=1, keepdims=True))
        dinv = 1.0 / jnp.maximum(deg, 1.0)
        h = jnp.maximum(agg * dinv + xr_ref[...], 0.0)
        hcat = jnp.dot(h, w_ref[...],
                       preferred_element_type=jnp.float32,
                       precision=lax.Precision.DEFAULT) + b_ref[...]
        o1_ref[0] = hcat[:, :dh]
        o1_ref[1] = hcat[:, dh:d]
        o2_ref[0] = hcat[:, d:d + dh]
        o2_ref[1] = hcat[:, d + dh:]
        o3_ref[...] = jnp.broadcast_to(dinv, (RB, LDEG))

    return pl.pallas_call(
        body,
        grid=(n // RB,),
        in_specs=[pl.BlockSpec((NC, RB, dha), lambda i: (0, i, 0)),
                  pl.BlockSpec((NC, RB, LDEG), lambda i: (0, i, 0)),
                  pl.BlockSpec((RB, d_in), lambda i: (i, 0)),
                  pl.BlockSpec(wcat.shape, lambda i: (0, 0)),
                  pl.BlockSpec(bcat.shape, lambda i: (0, 0))],
        out_specs=[pl.BlockSpec((NC, RB, dh), lambda i: (0, i, 0)),
                   pl.BlockSpec((NC, RB, dh), lambda i: (0, i, 0)),
                   pl.BlockSpec((RB, LDEG), lambda i: (i, 0))],
        out_shape=[jax.ShapeDtypeStruct((NC, n, dh), jnp.float32),
                   jax.ShapeDtypeStruct((NC, n, dh), jnp.float32),
                   jax.ShapeDtypeStruct((n, LDEG), jnp.float32)],
    )(aggp, degp, xr, wcat, bcat)


def kernel(x, edge_index, W1_l, W1_r, b1, W2_l, W2_r, b2):
    n, d_in = x.shape
    e = edge_index.shape[1]
    d_hid = W1_l.shape[0]
    d_out = W2_l.shape[0]

    ei = edge_index.astype(jnp.int32)
    grp = K * NSLOT
    ewp = -(-(-(-e // NS)) // grp) * grp
    ep = ewp * NS
    pad = ep - e
    if pad:
        src_flat = jnp.concatenate([ei[0], jnp.zeros((pad,), jnp.int32)])
        dst_flat = jnp.concatenate([ei[1], jnp.full((pad,), n, jnp.int32)])
    else:
        src_flat = ei[0]
        dst_flat = ei[1]
    nch = ewp // K
    src2 = src_flat.reshape(NS, nch, K)
    srcw = jnp.concatenate([src2, src2 + n], axis=0)
    dstw = dst_flat.reshape(NS, nch, K)

    w1cat = jnp.concatenate([W1_l.T, W1_r.T], axis=1)
    b1cat = jnp.concatenate([jnp.zeros_like(b1), b1]).reshape(1, 2 * d_hid)
    w2cat = jnp.concatenate([W2_l.T, W2_r.T], axis=1)
    b2cat = jnp.concatenate([jnp.zeros_like(b2), b2]).reshape(1, 2 * d_out)

    agg1 = _make_sc_aggregate(n, d_hid, ep, with_deg=True, fused_tail=False)
    agg2 = _make_sc_aggregate(n, d_out, ep, with_deg=False, fused_tail=True)

    xl, xr = _mm_in(x, w1cat, b1cat, d_hid)
    aggp1, degp = agg1(xl.reshape(NC * n, d_hid // NC), srcw, dstw)
    hl, hrp, dinvp = _mid(aggp1, degp, xr, w2cat, b2cat, d_out)
    out = agg2(hl.reshape(NC * n, d_out // NC), srcw, dstw, hrp, dinvp)
    if isinstance(out, (list, tuple)):
        out = out[0]
    return out


# speedup vs baseline: 1.8247x; 1.2822x over previous
import jax
import jax.numpy as jnp
from jax import lax
from jax.experimental import pallas as pl
from jax.experimental.pallas import tpu as pltpu
from jax.experimental.pallas import tpu_sc as plsc

NC = 2
NS = 16
LANES = 16
NW = NC * NS

LDEG = 16
ZB = 80
K = 80
NSLOT = 5
LOOKAHEAD = 3
RB = 2000


def _make_sc_aggregate(n, d, e, with_deg, fused_tail):
    dh = d // NC
    ew = e // NS
    k = K
    nch = ew // k
    nb = n // ZB
    tmax = -(-nb // NS)

    mesh = plsc.VectorSubcoreMesh(core_axis_name="c", subcore_axis_name="s")

    out_type = [jax.ShapeDtypeStruct((n, d), jnp.float32)]
    scratch = [
        pltpu.VMEM((nch, k), jnp.int32),
        pltpu.VMEM((nch, k), jnp.int32),
        pltpu.VMEM((NSLOT, k, dh), jnp.float32),
        pltpu.VMEM((ZB, dh), jnp.float32),
        pltpu.SemaphoreType.DMA((NSLOT,)),
        pltpu.SemaphoreType.DMA((NSLOT,)),
        pltpu.SemaphoreType.DMA((2,)),
        pltpu.VMEM_SHARED((n + 8, dh), jnp.float32),
    ]
    if with_deg:
        out_type.append(jax.ShapeDtypeStruct((n, d), jnp.float32))
        scratch += [
            pltpu.VMEM((k, LDEG), jnp.float32),
            pltpu.VMEM((ZB, LDEG), jnp.float32),
            pltpu.VMEM_SHARED((n + 8, LDEG), jnp.float32),
        ]
    if fused_tail:
        scratch += [
            pltpu.VMEM((ZB, dh), jnp.float32),
            pltpu.VMEM((ZB, LDEG), jnp.float32),
        ]

    def body(table, eiw, *refs):
        hrp = dinvp = None
        if fused_tail:
            hrp, dinvp = refs[:2]
            refs = refs[2:]
        if with_deg:
            (out, dego, srcv, dstv, bufs, zbuf, gsem, ssem, isem, acc,
             ones, zdeg, dacc) = refs
        else:
            if fused_tail:
                (out, srcv, dstv, bufs, zbuf, gsem, ssem, isem, acc,
                 hbuf, dvbuf) = refs
            else:
                out, srcv, dstv, bufs, zbuf, gsem, ssem, isem, acc = refs
        c = lax.axis_index("c")
        s = lax.axis_index("s")
        z16 = jnp.zeros((LANES,), jnp.float32)

        pltpu.async_copy(eiw.at[0, s], srcv, isem.at[0])
        pltpu.async_copy(eiw.at[1, s], dstv, isem.at[1])

        @pl.loop(0, ZB)
        def _(r):
            @pl.loop(0, dh, step=LANES)
            def _(cc):
                zbuf[r, pl.ds(cc, LANES)] = z16

        @pl.loop(0, tmax)
        def _(t):
            bid = s + t * NS

            @pl.when(bid < nb)
            def _():
                pltpu.sync_copy(zbuf, acc.at[pl.ds(bid * ZB, ZB)])

        if with_deg:
            o16 = jnp.ones((LANES,), jnp.float32)

            @pl.loop(0, ZB)
            def _(r):
                zdeg[r, pl.ds(0, LDEG)] = z16

            @pl.loop(0, k)
            def _(r):
                ones[r, pl.ds(0, LDEG)] = o16

            @pl.loop(0, tmax)
            def _(t):
                bid = s + t * NS

                @pl.when(bid < nb)
                def _():
                    pltpu.sync_copy(zdeg, dacc.at[pl.ds(bid * ZB, ZB)])

        pltpu.make_async_copy(eiw.at[0, s], srcv, isem.at[0]).wait()
        pltpu.make_async_copy(eiw.at[1, s], dstv, isem.at[1]).wait()

        @pl.loop(0, nch)
        def _(r):
            for q in range(k // LANES):
                sl = pl.ds(q * LANES, LANES)
                srcv[r, sl] = srcv[r, sl] * 2 + c

        plsc.subcore_barrier()

        def start_gather(i, b):
            pltpu.async_copy(table.at[srcv.at[i]], bufs.at[b], gsem.at[b])

        def wait_gather(i, b):
            pltpu.make_async_copy(table.at[srcv.at[i]], bufs.at[b],
                                  gsem.at[b]).wait()

        def start_scatter(i, b):
            pltpu.async_copy(bufs.at[b], acc.at[dstv.at[i]], ssem.at[b],
                             add=True)

        def wait_scatter(i, b):
            pltpu.make_async_copy(bufs.at[b], acc.at[dstv.at[i]],
                                  ssem.at[b]).wait()

        for b in range(LOOKAHEAD):
            start_gather(b, b)

        @pl.loop(0, nch, step=NSLOT)
        def _(j):
            for b in range(NSLOT):
                i = j + b
                bl = (b + LOOKAHEAD) % NSLOT

                @pl.when(i + LOOKAHEAD < nch)
                def _():
                    @pl.when(i + LOOKAHEAD - NSLOT >= 0)
                    def _():
                        wait_scatter(i + LOOKAHEAD - NSLOT, bl)

                    start_gather(i + LOOKAHEAD, bl)

                wait_gather(i, b)
                start_scatter(i, b)
                if with_deg:
                    @pl.when(lax.rem(i, 2) == c)
                    def _():
                        pltpu.sync_copy(ones, dacc.at[dstv.at[i]], add=True)

        for b in range(NSLOT):
            i = nch - NSLOT + b
            wait_scatter(i, b)

        plsc.subcore_barrier()

        if fused_tail:
            @pl.loop(0, tmax)
            def _(t):
                bid = s + t * NS

                @pl.when(bid < nb)
                def _():
                    r0 = bid * ZB
                    pltpu.sync_copy(acc.at[pl.ds(r0, ZB)], zbuf)
                    pltpu.sync_copy(
                        hrp.at[pl.ds(r0, ZB), pl.ds(c * dh, dh)], hbuf)
                    pltpu.sync_copy(
                        dinvp.at[pl.ds(r0, ZB), pl.ds(0, LDEG)], dvbuf)

                    @pl.loop(0, ZB)
                    def _(r):
                        dinv = dvbuf[r, pl.ds(0, LANES)]
                        for q in range(dh // LANES):
                            sl = pl.ds(q * LANES, LANES)
                            zbuf[r, sl] = zbuf[r, sl] * dinv + hbuf[r, sl]

                    pltpu.sync_copy(
                        zbuf, out.at[pl.ds(r0, ZB), pl.ds(c * dh, dh)])
        else:
            @pl.loop(0, tmax)
            def _(t):
                bid = s + t * NS

                @pl.when(bid < nb)
                def _():
                    pltpu.sync_copy(
                        acc.at[pl.ds(bid * ZB, ZB)],
                        out.at[pl.ds(bid * ZB, ZB), pl.ds(c * dh, dh)])

        if with_deg:
            @pl.loop(0, tmax)
            def _(t):
                bid = s + t * NS

                @pl.when(bid < nb)
                def _():
                    pltpu.sync_copy(
                        dacc.at[pl.ds(bid * ZB, ZB)],
                        dego.at[pl.ds(bid * ZB, ZB),
                                pl.ds(c * LDEG, LDEG)])

    return pl.kernel(
        body, out_type=out_type, mesh=mesh, scratch_types=scratch,
        compiler_params=pltpu.CompilerParams(use_tc_tiling_on_sc=False))


def _mm_in(x, wcat, bcat, d):
    n = x.shape[0]
    d_in = x.shape[1]

    def body(x_ref, w_ref, b_ref, o1_ref, o2_ref):
        h = jnp.dot(x_ref[...], w_ref[...],
                    preferred_element_type=jnp.float32,
                    precision=lax.Precision.DEFAULT) + b_ref[...]
        o1_ref[...] = h[:, :d]
        o2_ref[...] = h[:, d:]

    return pl.pallas_call(
        body,
        grid=(n // RB,),
        in_specs=[pl.BlockSpec((RB, d_in), lambda i: (i, 0)),
                  pl.BlockSpec(wcat.shape, lambda i: (0, 0)),
                  pl.BlockSpec(bcat.shape, lambda i: (0, 0))],
        out_specs=[pl.BlockSpec((RB, d), lambda i: (i, 0)),
                   pl.BlockSpec((RB, d), lambda i: (i, 0))],
        out_shape=[jax.ShapeDtypeStruct((n, d), jnp.float32),
                   jax.ShapeDtypeStruct((n, d), jnp.float32)],
    )(x, wcat, bcat)


def _mid(aggp, degp, xr, wcat, bcat, d):
    n, d_in = xr.shape

    def body(a_ref, g_ref, xr_ref, w_ref, b_ref, o1_ref, o2_ref, o3_ref):
        deg = (jnp.max(g_ref[:, :LDEG], axis=1, keepdims=True)
               + jnp.max(g_ref[:, LDEG:2 * LDEG], axis=1, keepdims=True))
        dinv = 1.0 / jnp.maximum(deg, 1.0)
        h = jnp.maximum(a_ref[...] * dinv + xr_ref[...], 0.0)
        hcat = jnp.dot(h, w_ref[...],
                       preferred_element_type=jnp.float32,
                       precision=lax.Precision.DEFAULT) + b_ref[...]
        o1_ref[...] = hcat[:, :d]
        o2_ref[...] = hcat[:, d:]
        o3_ref[...] = jnp.broadcast_to(dinv, (RB, d))

    return pl.pallas_call(
        body,
        grid=(n // RB,),
        in_specs=[pl.BlockSpec((RB, d_in), lambda i: (i, 0)),
                  pl.BlockSpec((RB, d_in), lambda i: (i, 0)),
                  pl.BlockSpec((RB, d_in), lambda i: (i, 0)),
                  pl.BlockSpec(wcat.shape, lambda i: (0, 0)),
                  pl.BlockSpec(bcat.shape, lambda i: (0, 0))],
        out_specs=[pl.BlockSpec((RB, d), lambda i: (i, 0)),
                   pl.BlockSpec((RB, d), lambda i: (i, 0)),
                   pl.BlockSpec((RB, d), lambda i: (i, 0))],
        out_shape=[jax.ShapeDtypeStruct((n, d), jnp.float32),
                   jax.ShapeDtypeStruct((n, d), jnp.float32),
                   jax.ShapeDtypeStruct((n, d), jnp.float32)],
    )(aggp, degp, xr, wcat, bcat)


def kernel(x, edge_index, W1_l, W1_r, b1, W2_l, W2_r, b2):
    n, d_in = x.shape
    e = edge_index.shape[1]
    d_hid = W1_l.shape[0]
    d_out = W2_l.shape[0]

    ei = edge_index.astype(jnp.int32)
    grp = K * NSLOT
    ewp = -(-(-(-e // NS)) // grp) * grp
    ep = ewp * NS
    pad = ep - e
    if pad:
        src_flat = jnp.concatenate([ei[0], jnp.zeros((pad,), jnp.int32)])
        dst_flat = jnp.concatenate([ei[1], jnp.full((pad,), n, jnp.int32)])
        eiw = jnp.stack([src_flat, dst_flat]).reshape(2, NS, ewp // K, K)
    else:
        eiw = ei.reshape(2, NS, ewp // K, K)

    w1cat = jnp.concatenate([W1_l.T, W1_r.T], axis=1)
    b1cat = jnp.concatenate([jnp.zeros_like(b1), b1]).reshape(1, 2 * d_hid)
    w2cat = jnp.concatenate([W2_l.T, W2_r.T], axis=1)
    b2cat = jnp.concatenate([jnp.zeros_like(b2), b2]).reshape(1, 2 * d_out)

    agg1 = _make_sc_aggregate(n, d_hid, ep, with_deg=True, fused_tail=False)
    agg2 = _make_sc_aggregate(n, d_out, ep, with_deg=False, fused_tail=True)

    xl, xr = _mm_in(x, w1cat, b1cat, d_hid)
    aggp1, degp = agg1(xl.reshape(NC * n, d_hid // NC), eiw)
    hl, hrp, dinvp = _mid(aggp1, degp, xr, w2cat, b2cat, d_out)
    out = agg2(hl.reshape(NC * n, d_out // NC), eiw, hrp, dinvp)
    if isinstance(out, (list, tuple)):
        out = out[0]
    return out
